# baseline (speedup 1.0000x reference)
import functools
import jax
import jax.numpy as jnp
from jax import lax
from jax.experimental import pallas as pl
from jax.experimental.pallas import tpu as pltpu
from jax.experimental.pallas import tpu_sc as plsc

B, T, D = 1, 2048, 768
H, KVH, DH = 16, 1, 64
G = H // KVH
BLK = 32
NSEL = 16
BALL = 128
W = T // BLK
NB = T // BALL
SCALE = DH ** -0.5
TQ = 256
NQT = T // TQ
HB = 4
NBT = TQ // BALL
NEG = -jnp.finfo(jnp.float32).max / 10.0
MBIG = 512.0

_INTERPRET = False


def _dotT(a, b):
    return jax.lax.dot_general(a, b, (((1,), (1,)), ((), ())),
                               preferred_element_type=jnp.float32)


def _dot(a, b):
    return jnp.dot(a, b, preferred_element_type=jnp.float32)


def _softmax_unnorm(s):
    m = jnp.max(s, axis=-1, keepdims=True)
    e = jnp.exp(s - m)
    denom = jnp.sum(e, axis=-1, keepdims=True)
    return e.astype(jnp.bfloat16), 1.0 / denom


def _make_sc_mask():
    info = plsc.get_sparse_core_info()
    NC, NS, L = info.num_cores, info.num_subcores, info.num_lanes
    QW = 128
    NWORK = T // QW
    mesh = plsc.VectorSubcoreMesh(core_axis_name="c", subcore_axis_name="s")

    @functools.partial(
        pl.kernel, mesh=mesh,
        out_type=jax.ShapeDtypeStruct((W, T), jnp.float32),
        scratch_types=[
            pltpu.VMEM((W, QW), jnp.float32),
            pltpu.VMEM((W, QW), jnp.float32),
        ],
    )
    def sc_mask(impT_hbm, maskT_hbm, impv, maskv):
        wid = lax.axis_index("s") * NC + lax.axis_index("c")

        @pl.when(wid < NWORK)
        def _():
            base = wid * QW
            pltpu.sync_copy(impT_hbm.at[:, pl.ds(base, QW)], impv)
            for vg in range(QW // L):
                sl = pl.ds(vg * L, L)

                def pass_body(p, carry):
                    t_val, t_idx = carry
                    m_val = jnp.full((L,), -jnp.inf, jnp.float32)
                    m_idx = jnp.full((L,), W, jnp.int32)
                    for j in range(W):
                        x = impv[j, sl]
                        jc = jnp.full((L,), j, jnp.int32)
                        elig = (x < t_val) | ((x == t_val) & (jc > t_idx))
                        better = elig & ((x > m_val)
                                         | ((x == m_val) & (jc < m_idx)))
                        m_val = jnp.where(better, x, m_val)
                        m_idx = jnp.where(better, jc, m_idx)
                    return m_val, m_idx

                t0 = (jnp.full((L,), jnp.inf, jnp.float32),
                      jnp.full((L,), -1, jnp.int32))
                t_val, t_idx = lax.fori_loop(0, NSEL, pass_body, t0)
                for j in range(W):
                    x = impv[j, sl]
                    jc = jnp.full((L,), j, jnp.int32)
                    sel = (x > t_val) | ((x == t_val) & (jc <= t_idx))
                    maskv[j, sl] = jnp.where(sel, jnp.float32(1.0),
                                             jnp.float32(0.0))
            pltpu.sync_copy(maskv, maskT_hbm.at[:, pl.ds(base, QW)])

    return sc_mask


def _qkv_kernel(x_ref, gamma_ref, wqkv_ref, wcomb_ref, bcomb_ref,
                qkv_ref, gate_ref):
    x = x_ref[...]
    eps = jnp.finfo(jnp.float32).eps
    xn = x * jax.lax.rsqrt(jnp.mean(x * x, axis=-1, keepdims=True) + eps)
    xn = xn * gamma_ref[...]
    qkv_ref[...] = _dot(xn, wqkv_ref[...])
    gate_ref[...] = jax.nn.sigmoid(_dot(xn, wcomb_ref[...]) + bcomb_ref[...])


def _compress_kernel(kw_ref, vw_ref, kW1_ref, kb1_ref, kW2_ref, kb2_ref,
                     vW1_ref, vb1_ref, vW2_ref, vb2_ref, memk_ref, memv_ref,
                     ck_ref, cv_ref):
    hk = jnp.maximum(_dot(kw_ref[...], kW1_ref[...]) + kb1_ref[...], 0.0)
    ck = _dot(hk, kW2_ref[...]) + kb2_ref[...]
    hv = jnp.maximum(_dot(vw_ref[...], vW1_ref[...]) + vb1_ref[...], 0.0)
    cv = _dot(hv, vW2_ref[...]) + vb2_ref[...]
    ck_ref[...] = jnp.zeros((2 * W, DH), jnp.float32)
    cv_ref[...] = jnp.zeros((2 * W, DH), jnp.float32)
    ck_ref[0:W, :] = ck
    cv_ref[0:W, :] = cv
    ck_ref[W:W + 1, :] = memk_ref[...]
    cv_ref[W:W + 1, :] = memv_ref[...]


def _cattn_kernel(q_ref, ck_ref, cv_ref, g_ref, co_ref, impT_ref):
    q = q_ref[...].reshape(H * TQ, DH)
    s = _dotT(q, ck_ref[...]) * SCALE
    col = jax.lax.broadcasted_iota(jnp.int32, s.shape, 1)
    sm = jnp.where(col <= W, s, NEG)
    e, rdenom = _softmax_unnorm(sm)
    co = (_dot(e, cv_ref[...]) * rdenom * g_ref[...].reshape(H * TQ, 1)
          ).reshape(H, TQ, DH)
    co_ref[...] = co
    imp = jnp.mean(s.reshape(H, TQ, 2 * W)[:, :, 0:W], axis=0)
    impT_ref[...] = jnp.transpose(imp)


def _ball_kernel(q_ref, kloc_ref, vloc_ref, pos_ref, sigma_ref, g2_ref,
                 so_ref):
    q_all = q_ref[...]
    pos = pos_ref[...]
    g2 = g2_ref[...]
    for bi in range(NBT):
        pb = pos[bi * BALL:(bi + 1) * BALL]
        pt = jnp.transpose(pb)
        d2 = jnp.zeros((BALL, BALL), jnp.float32)
        for ci in range(3):
            diff = pb[:, ci:ci + 1] - pt[ci:ci + 1, :]
            d2 = d2 + diff * diff
        dist = jnp.sqrt(jnp.maximum(d2, 0.0))
        sig = jnp.broadcast_to(sigma_ref[...], (H, BALL, 1)
                               ).reshape(H * BALL, 1)
        bias = sig * jnp.broadcast_to(dist[None], (H, BALL, BALL)
                                      ).reshape(H * BALL, BALL)
        qb = q_all[:, bi * BALL:(bi + 1) * BALL, :].reshape(H * BALL, DH)
        sb = _dotT(qb * SCALE, kloc_ref[bi * BALL:(bi + 1) * BALL, :]) + bias
        eb, rdb = _softmax_unnorm(sb)
        ob = _dot(eb, vloc_ref[bi * BALL:(bi + 1) * BALL, :]) * rdb
        ob = ob * g2[:, bi * BALL:(bi + 1) * BALL, :].reshape(H * BALL, 1)
        so_ref[:, bi * BALL:(bi + 1) * BALL, :] = ob.reshape(H, BALL, DH)


def _fine_kernel(q_ref, maskT_ref, co_ref, so_ref, kx_ref, vb_ref,
                 g1_ref, wout_ref, out_ref):
    q_all = q_ref[...]
    mtile = jnp.transpose(maskT_ref[...]) * MBIG
    so = so_ref[...]
    g1 = g1_ref[...]
    co = co_ref[...]
    kxb = kx_ref[...]
    acc = jnp.zeros((TQ, D), jnp.float32)
    for hg in range(H // HB):
        qh = q_all[hg * HB:(hg + 1) * HB].reshape(HB * TQ, DH) * SCALE
        qa = jnp.concatenate(
            [qh, jnp.broadcast_to(mtile[None], (HB, TQ, W)
                                  ).reshape(HB * TQ, W)], axis=1
        ).astype(jnp.bfloat16)
        sf = _dotT(qa, kxb)
        ef, rdf = _softmax_unnorm(sf.reshape(HB, TQ, T))
        fine = _dot(ef.reshape(HB * TQ, T), vb_ref[...]).reshape(HB, TQ, DH)
        mixed = (fine * rdf * g1[hg * HB:(hg + 1) * HB]
                 + co[hg * HB:(hg + 1) * HB] + so[hg * HB:(hg + 1) * HB])
        for j in range(HB):
            hh = hg * HB + j
            acc = acc + _dot(mixed[j], wout_ref[hh * DH:(hh + 1) * DH, :])
    out_ref[...] = acc


def kernel(inp, pos, gamma, Wqkv, mem_kv, kW1, kb1, kW2, kb2,
           vW1, vb1, vW2, vb2, sigma_att, Wcomb, bcomb, Wout):
    x = inp.reshape(T, D)
    NQKV = H * DH + 2 * KVH * DH

    qkv, gate = pl.pallas_call(
        _qkv_kernel,
        grid=(NQT,),
        in_specs=[
            pl.BlockSpec((TQ, D), lambda i: (i, 0)),
            pl.BlockSpec((1, D), lambda i: (0, 0)),
            pl.BlockSpec((D, NQKV), lambda i: (0, 0)),
            pl.BlockSpec((D, 3 * H), lambda i: (0, 0)),
            pl.BlockSpec((1, 3 * H), lambda i: (0, 0)),
        ],
        out_specs=[
            pl.BlockSpec((TQ, NQKV), lambda i: (i, 0)),
            pl.BlockSpec((TQ, 3 * H), lambda i: (i, 0)),
        ],
        out_shape=[
            jax.ShapeDtypeStruct((T, NQKV), jnp.float32),
            jax.ShapeDtypeStruct((T, 3 * H), jnp.float32),
        ],
        interpret=_INTERPRET,
    )(x, gamma.reshape(1, D), Wqkv, Wcomb, bcomb.reshape(1, 3 * H))

    k = jax.lax.slice(qkv, (0, H * DH), (T, H * DH + DH))
    v = jax.lax.slice(qkv, (0, H * DH + DH), (T, H * DH + 2 * DH))
    kw = k.reshape(W, BLK * DH)
    vw = v.reshape(W, BLK * DH)
    vb = v.astype(jnp.bfloat16)
    expandT = (jnp.arange(T)[:, None] // BLK
               == jnp.arange(W)[None, :]).astype(jnp.float32)
    kx = jnp.concatenate([k, expandT], axis=1).astype(jnp.bfloat16)
    q4 = qkv[:, :H * DH].reshape(T, H, DH).transpose(1, 0, 2)
    g0, g1, g2 = (gate[:, j::3].T.reshape(H, T, 1) for j in (0, 1, 2))

    full = lambda shape: pl.BlockSpec(shape, lambda: tuple(0 for _ in shape))
    ck, cv = pl.pallas_call(
        _compress_kernel,
        in_specs=[
            full((W, BLK * DH)), full((W, BLK * DH)),
            full((BLK * DH, BLK * DH)), full((1, BLK * DH)),
            full((BLK * DH, DH)), full((1, DH)),
            full((BLK * DH, BLK * DH)), full((1, BLK * DH)),
            full((BLK * DH, DH)), full((1, DH)),
            full((1, DH)), full((1, DH)),
        ],
        out_specs=[full((2 * W, DH)), full((2 * W, DH))],
        out_shape=[
            jax.ShapeDtypeStruct((2 * W, DH), jnp.float32),
            jax.ShapeDtypeStruct((2 * W, DH), jnp.float32),
        ],
        interpret=_INTERPRET,
    )(kw, vw, kW1, kb1.reshape(1, -1), kW2, kb2.reshape(1, -1),
      vW1, vb1.reshape(1, -1), vW2, vb2.reshape(1, -1),
      mem_kv[0, 0], mem_kv[1, 0])

    co4, impT = pl.pallas_call(
        _cattn_kernel,
        grid=(NQT,),
        in_specs=[
            pl.BlockSpec((H, TQ, DH), lambda i: (0, i, 0)),
            pl.BlockSpec((2 * W, DH), lambda i: (0, 0)),
            pl.BlockSpec((2 * W, DH), lambda i: (0, 0)),
            pl.BlockSpec((H, TQ, 1), lambda i: (0, i, 0)),
        ],
        out_specs=[
            pl.BlockSpec((H, TQ, DH), lambda i: (0, i, 0)),
            pl.BlockSpec((W, TQ), lambda i: (0, i)),
        ],
        out_shape=[
            jax.ShapeDtypeStruct((H, T, DH), jnp.float32),
            jax.ShapeDtypeStruct((W, T), jnp.float32),
        ],
        interpret=_INTERPRET,
    )(q4, ck, cv.astype(jnp.bfloat16), g0)

    maskT = _make_sc_mask()(impT)

    posp = jnp.pad(pos, ((0, 0), (0, 8 - pos.shape[1])))
    sigma3 = sigma_att.reshape(H, 1, 1)
    so4 = pl.pallas_call(
        _ball_kernel,
        grid=(NQT,),
        in_specs=[
            pl.BlockSpec((H, TQ, DH), lambda i: (0, i, 0)),
            pl.BlockSpec((TQ, DH), lambda i: (i, 0)),
            pl.BlockSpec((TQ, DH), lambda i: (i, 0)),
            pl.BlockSpec((TQ, 8), lambda i: (i, 0)),
            pl.BlockSpec((H, 1, 1), lambda i: (0, 0, 0)),
            pl.BlockSpec((H, TQ, 1), lambda i: (0, i, 0)),
        ],
        out_specs=pl.BlockSpec((H, TQ, DH), lambda i: (0, i, 0)),
        out_shape=jax.ShapeDtypeStruct((H, T, DH), jnp.float32),
        interpret=_INTERPRET,
    )(q4, k, vb, posp, sigma3, g2)

    out = pl.pallas_call(
        _fine_kernel,
        grid=(NQT,),
        in_specs=[
            pl.BlockSpec((H, TQ, DH), lambda i: (0, i, 0)),
            pl.BlockSpec((W, TQ), lambda i: (0, i)),
            pl.BlockSpec((H, TQ, DH), lambda i: (0, i, 0)),
            pl.BlockSpec((H, TQ, DH), lambda i: (0, i, 0)),
            pl.BlockSpec((T, DH + W), lambda i: (0, 0)),
            pl.BlockSpec((T, DH), lambda i: (0, 0)),
            pl.BlockSpec((H, TQ, 1), lambda i: (0, i, 0)),
            pl.BlockSpec((H * DH, D), lambda i: (0, 0)),
        ],
        out_specs=pl.BlockSpec((TQ, D), lambda i: (i, 0)),
        out_shape=jax.ShapeDtypeStruct((T, D), jnp.float32),
        interpret=_INTERPRET,
    )(q4, maskT, co4, so4, kx, vb, g1, Wout)

    return out.reshape(B, T, D)


# speedup vs baseline: 1.0932x; 1.0932x over previous
import functools
import jax
import jax.numpy as jnp
from jax import lax
from jax.experimental import pallas as pl
from jax.experimental.pallas import tpu as pltpu
from jax.experimental.pallas import tpu_sc as plsc

B, T, D = 1, 2048, 768
H, KVH, DH = 16, 1, 64
G = H // KVH
BLK = 32
NSEL = 16
BALL = 128
W = T // BLK
NB = T // BALL
SCALE = DH ** -0.5
TQ = 256
NQT = T // TQ
HB = 4
NBT = TQ // BALL
NEG = -jnp.finfo(jnp.float32).max / 10.0
MBIG = 512.0

_INTERPRET = False


def _dotT(a, b):
    return jax.lax.dot_general(a, b, (((1,), (1,)), ((), ())),
                               preferred_element_type=jnp.float32)


def _dot(a, b):
    return jnp.dot(a, b, preferred_element_type=jnp.float32)


def _softmax_unnorm(s):
    m = jnp.max(s, axis=-1, keepdims=True)
    e = jnp.exp(s - m)
    denom = jnp.sum(e, axis=-1, keepdims=True)
    return e.astype(jnp.bfloat16), 1.0 / denom


def _make_sc_mask():
    info = plsc.get_sparse_core_info()
    NC, NS, L = info.num_cores, info.num_subcores, info.num_lanes
    QW = 128
    NWORK = T // QW
    mesh = plsc.VectorSubcoreMesh(core_axis_name="c", subcore_axis_name="s")

    @functools.partial(
        pl.kernel, mesh=mesh,
        out_type=jax.ShapeDtypeStruct((W, T), jnp.float32),
        scratch_types=[
            pltpu.VMEM((W, QW), jnp.float32),
            pltpu.VMEM((W, QW), jnp.float32),
        ],
    )
    def sc_mask(impT_hbm, maskT_hbm, impv, maskv):
        wid = lax.axis_index("s") * NC + lax.axis_index("c")

        @pl.when(wid < NWORK)
        def _():
            base = wid * QW
            pltpu.sync_copy(impT_hbm.at[:, pl.ds(base, QW)], impv)
            for vg in range(QW // L):
                sl = pl.ds(vg * L, L)

                def pass_body(p, carry):
                    t_val, t_idx = carry
                    m_val = jnp.full((L,), -jnp.inf, jnp.float32)
                    m_idx = jnp.full((L,), W, jnp.int32)
                    for j in range(W):
                        x = impv[j, sl]
                        jc = jnp.full((L,), j, jnp.int32)
                        elig = (x < t_val) | ((x == t_val) & (jc > t_idx))
                        better = elig & ((x > m_val)
                                         | ((x == m_val) & (jc < m_idx)))
                        m_val = jnp.where(better, x, m_val)
                        m_idx = jnp.where(better, jc, m_idx)
                    return m_val, m_idx

                t0 = (jnp.full((L,), jnp.inf, jnp.float32),
                      jnp.full((L,), -1, jnp.int32))
                t_val, t_idx = lax.fori_loop(0, NSEL, pass_body, t0)
                for j in range(W):
                    x = impv[j, sl]
                    jc = jnp.full((L,), j, jnp.int32)
                    sel = (x > t_val) | ((x == t_val) & (jc <= t_idx))
                    maskv[j, sl] = jnp.where(sel, jnp.float32(1.0),
                                             jnp.float32(0.0))
            pltpu.sync_copy(maskv, maskT_hbm.at[:, pl.ds(base, QW)])

    return sc_mask


def _qkv_kernel(x_ref, gamma_ref, wqkv_ref, wcomb_ref, bcomb_ref,
                qkv_ref, gate_ref):
    x = x_ref[...]
    eps = jnp.finfo(jnp.float32).eps
    xn = x * jax.lax.rsqrt(jnp.mean(x * x, axis=-1, keepdims=True) + eps)
    xn = xn * gamma_ref[...]
    qkv_ref[...] = _dot(xn, wqkv_ref[...])
    gate_ref[...] = jax.nn.sigmoid(_dot(xn, wcomb_ref[...]) + bcomb_ref[...])


def _bc1_kernel(kw_ref, vw_ref, kW1_ref, kb1_ref, kW2_ref, kb2_ref,
                vW1_ref, vb1_ref, vW2_ref, vb2_ref, memk_ref, memv_ref,
                q_ref, g_ref, ck_ref, cvb_ref, co_ref, impT_ref):
    i = pl.program_id(0)

    @pl.when(i == 0)
    def _b():
        hk = jnp.maximum(_dot(kw_ref[...], kW1_ref[...]) + kb1_ref[...], 0.0)
        ck = _dot(hk, kW2_ref[...]) + kb2_ref[...]
        hv = jnp.maximum(_dot(vw_ref[...], vW1_ref[...]) + vb1_ref[...], 0.0)
        cv = _dot(hv, vW2_ref[...]) + vb2_ref[...]
        ck_ref[...] = jnp.zeros((2 * W, DH), jnp.float32)
        cvb_ref[...] = jnp.zeros((2 * W, DH), jnp.bfloat16)
        ck_ref[0:W, :] = ck
        cvb_ref[0:W, :] = cv.astype(jnp.bfloat16)
        ck_ref[W:W + 1, :] = memk_ref[...]
        cvb_ref[W:W + 1, :] = memv_ref[...].astype(jnp.bfloat16)

    @pl.when(i > 0)
    def _c1():
        q = q_ref[...].reshape(H * TQ, DH)
        s = _dotT(q, ck_ref[...]) * SCALE
        col = jax.lax.broadcasted_iota(jnp.int32, s.shape, 1)
        sm = jnp.where(col <= W, s, NEG)
        e, rdenom = _softmax_unnorm(sm)
        co = (_dot(e, cvb_ref[...]) * rdenom * g_ref[...].reshape(H * TQ, 1)
              ).reshape(H, TQ, DH)
        co_ref[...] = co
        imp = jnp.mean(s.reshape(H, TQ, 2 * W)[:, :, 0:W], axis=0)
        impT_ref[...] = jnp.transpose(imp)


def _mega2_kernel(q_ref, maskT_ref, co_ref, kx_ref, vb_ref, kloc_ref,
                  vloc_ref, pos_ref, sigma_ref, g1_ref, g2_ref, wout_ref,
                  out_ref):
    q_all = q_ref[...]
    mtile = jnp.transpose(maskT_ref[...]) * MBIG

    pos = pos_ref[...]
    g2 = g2_ref[...]
    ball_outs = []
    for bi in range(NBT):
        pb = pos[bi * BALL:(bi + 1) * BALL]
        pt = jnp.transpose(pb)
        d2 = jnp.zeros((BALL, BALL), jnp.float32)
        for ci in range(3):
            diff = pb[:, ci:ci + 1] - pt[ci:ci + 1, :]
            d2 = d2 + diff * diff
        dist = jnp.sqrt(jnp.maximum(d2, 0.0))
        sig = jnp.broadcast_to(sigma_ref[...], (H, BALL, 1)
                               ).reshape(H * BALL, 1)
        bias = sig * jnp.broadcast_to(dist[None], (H, BALL, BALL)
                                      ).reshape(H * BALL, BALL)
        qb = q_all[:, bi * BALL:(bi + 1) * BALL, :].reshape(H * BALL, DH)
        sb = _dotT(qb * SCALE, kloc_ref[bi * BALL:(bi + 1) * BALL, :]) + bias
        eb, rdb = _softmax_unnorm(sb)
        ob = _dot(eb, vloc_ref[bi * BALL:(bi + 1) * BALL, :]) * rdb
        ob = ob * g2[:, bi * BALL:(bi + 1) * BALL, :].reshape(H * BALL, 1)
        ball_outs.append(ob.reshape(H, BALL, DH))
    so = jnp.concatenate(ball_outs, axis=1)

    g1 = g1_ref[...]
    co = co_ref[...]
    acc = jnp.zeros((TQ, D), jnp.float32)
    for hg in range(H // HB):
        qh = q_all[hg * HB:(hg + 1) * HB].reshape(HB * TQ, DH) * SCALE
        qa = jnp.concatenate(
            [qh, jnp.broadcast_to(mtile[None], (HB, TQ, W)
                                  ).reshape(HB * TQ, W)], axis=1)
        sf = _dotT(qa, kx_ref[...])
        ef, rdf = _softmax_unnorm(sf.reshape(HB, TQ, T))
        fine = _dot(ef.reshape(HB * TQ, T), vb_ref[...]).reshape(HB, TQ, DH)
        mixed = (fine * rdf * g1[hg * HB:(hg + 1) * HB]
                 + co[hg * HB:(hg + 1) * HB] + so[hg * HB:(hg + 1) * HB])
        for j in range(HB):
            hh = hg * HB + j
            acc = acc + _dot(mixed[j], wout_ref[hh * DH:(hh + 1) * DH, :])
    out_ref[...] = acc


def kernel(inp, pos, gamma, Wqkv, mem_kv, kW1, kb1, kW2, kb2,
           vW1, vb1, vW2, vb2, sigma_att, Wcomb, bcomb, Wout):
    x = inp.reshape(T, D)
    NQKV = H * DH + 2 * KVH * DH

    qkv, gate = pl.pallas_call(
        _qkv_kernel,
        grid=(NQT,),
        in_specs=[
            pl.BlockSpec((TQ, D), lambda i: (i, 0)),
            pl.BlockSpec((1, D), lambda i: (0, 0)),
            pl.BlockSpec((D, NQKV), lambda i: (0, 0)),
            pl.BlockSpec((D, 3 * H), lambda i: (0, 0)),
            pl.BlockSpec((1, 3 * H), lambda i: (0, 0)),
        ],
        out_specs=[
            pl.BlockSpec((TQ, NQKV), lambda i: (i, 0)),
            pl.BlockSpec((TQ, 3 * H), lambda i: (i, 0)),
        ],
        out_shape=[
            jax.ShapeDtypeStruct((T, NQKV), jnp.float32),
            jax.ShapeDtypeStruct((T, 3 * H), jnp.float32),
        ],
        interpret=_INTERPRET,
    )(x, gamma.reshape(1, D), Wqkv, Wcomb, bcomb.reshape(1, 3 * H))

    k = jax.lax.slice(qkv, (0, H * DH), (T, H * DH + DH))
    v = jax.lax.slice(qkv, (0, H * DH + DH), (T, H * DH + 2 * DH))
    kw = k.reshape(W, BLK * DH)
    vw = v.reshape(W, BLK * DH)
    vb = v.astype(jnp.bfloat16)
    expandT = (jnp.arange(T)[:, None] // BLK
               == jnp.arange(W)[None, :]).astype(jnp.float32)
    kx = jnp.concatenate([k, expandT], axis=1)
    q4 = qkv[:, :H * DH].reshape(T, H, DH).transpose(1, 0, 2)
    g0, g1, g2 = (gate[:, j::3].T.reshape(H, T, 1) for j in (0, 1, 2))

    full = lambda shape: pl.BlockSpec(shape, lambda i: tuple(0 for _ in shape))
    tile = lambda i: (0, jnp.maximum(i - 1, 0), 0)
    _, _, co4, impT = pl.pallas_call(
        _bc1_kernel,
        grid=(1 + NQT,),
        in_specs=[
            full((W, BLK * DH)), full((W, BLK * DH)),
            full((BLK * DH, BLK * DH)), full((1, BLK * DH)),
            full((BLK * DH, DH)), full((1, DH)),
            full((BLK * DH, BLK * DH)), full((1, BLK * DH)),
            full((BLK * DH, DH)), full((1, DH)),
            full((1, DH)), full((1, DH)),
            pl.BlockSpec((H, TQ, DH), tile),
            pl.BlockSpec((H, TQ, 1), tile),
        ],
        out_specs=[
            full((2 * W, DH)), full((2 * W, DH)),
            pl.BlockSpec((H, TQ, DH), tile),
            pl.BlockSpec((W, TQ), lambda i: (0, jnp.maximum(i - 1, 0))),
        ],
        out_shape=[
            jax.ShapeDtypeStruct((2 * W, DH), jnp.float32),
            jax.ShapeDtypeStruct((2 * W, DH), jnp.bfloat16),
            jax.ShapeDtypeStruct((H, T, DH), jnp.float32),
            jax.ShapeDtypeStruct((W, T), jnp.float32),
        ],
        interpret=_INTERPRET,
    )(kw, vw, kW1, kb1.reshape(1, -1), kW2, kb2.reshape(1, -1),
      vW1, vb1.reshape(1, -1), vW2, vb2.reshape(1, -1),
      mem_kv[0, 0], mem_kv[1, 0], q4, g0)

    maskT = _make_sc_mask()(impT)

    posp = jnp.pad(pos, ((0, 0), (0, 8 - pos.shape[1])))
    sigma3 = sigma_att.reshape(H, 1, 1)
    out = pl.pallas_call(
        _mega2_kernel,
        grid=(NQT,),
        in_specs=[
            pl.BlockSpec((H, TQ, DH), lambda i: (0, i, 0)),
            pl.BlockSpec((W, TQ), lambda i: (0, i)),
            pl.BlockSpec((H, TQ, DH), lambda i: (0, i, 0)),
            pl.BlockSpec((T, DH + W), lambda i: (0, 0)),
            pl.BlockSpec((T, DH), lambda i: (0, 0)),
            pl.BlockSpec((TQ, DH), lambda i: (i, 0)),
            pl.BlockSpec((TQ, DH), lambda i: (i, 0)),
            pl.BlockSpec((TQ, 8), lambda i: (i, 0)),
            pl.BlockSpec((H, 1, 1), lambda i: (0, 0, 0)),
            pl.BlockSpec((H, TQ, 1), lambda i: (0, i, 0)),
            pl.BlockSpec((H, TQ, 1), lambda i: (0, i, 0)),
            pl.BlockSpec((H * DH, D), lambda i: (0, 0)),
        ],
        out_specs=pl.BlockSpec((TQ, D), lambda i: (i, 0)),
        out_shape=jax.ShapeDtypeStruct((T, D), jnp.float32),
        interpret=_INTERPRET,
    )(q4, maskT, co4, kx, vb, k, vb, posp, sigma3, g1, g2, Wout)

    return out.reshape(B, T, D)


# speedup vs baseline: 1.0941x; 1.0008x over previous
import functools
import jax
import jax.numpy as jnp
from jax import lax
from jax.experimental import pallas as pl
from jax.experimental.pallas import tpu as pltpu
from jax.experimental.pallas import tpu_sc as plsc

B, T, D = 1, 2048, 768
H, KVH, DH = 16, 1, 64
G = H // KVH
BLK = 32
NSEL = 16
BALL = 128
W = T // BLK
NB = T // BALL
SCALE = DH ** -0.5
TQ = 256
NQT = T // TQ
HB = 4
NBT = TQ // BALL
NEG = -jnp.finfo(jnp.float32).max / 10.0
MBIG = 512.0

_INTERPRET = False


def _dotT(a, b):
    return jax.lax.dot_general(a, b, (((1,), (1,)), ((), ())),
                               preferred_element_type=jnp.float32)


def _dot(a, b):
    return jnp.dot(a, b, preferred_element_type=jnp.float32)


def _softmax_unnorm(s):
    m = jnp.max(s, axis=-1, keepdims=True)
    e = jnp.exp(s - m)
    denom = jnp.sum(e, axis=-1, keepdims=True)
    return e.astype(jnp.bfloat16), 1.0 / denom


def _make_sc_mask():
    info = plsc.get_sparse_core_info()
    NC, NS, L = info.num_cores, info.num_subcores, info.num_lanes
    NWORK = NC * NS
    QW = T // NWORK
    mesh = plsc.VectorSubcoreMesh(core_axis_name="c", subcore_axis_name="s")

    @functools.partial(
        pl.kernel, mesh=mesh,
        out_type=jax.ShapeDtypeStruct((NWORK, W, QW), jnp.float32),
        scratch_types=[
            pltpu.VMEM((W, QW), jnp.float32),
            pltpu.VMEM((W, QW), jnp.float32),
        ],
    )
    def sc_mask(imp3_hbm, mask3_hbm, impv, maskv):
        wid = lax.axis_index("s") * NC + lax.axis_index("c")

        @pl.when(wid < NWORK)
        def _():
            pltpu.sync_copy(imp3_hbm.at[wid], impv)
            for vg in range(QW // L):
                sl = pl.ds(vg * L, L)

                def pass_body(p, carry):
                    t_val, t_idx = carry
                    m_val = jnp.full((L,), -jnp.inf, jnp.float32)
                    m_idx = jnp.full((L,), W, jnp.int32)
                    for j in range(W):
                        x = impv[j, sl]
                        jc = jnp.full((L,), j, jnp.int32)
                        elig = (x < t_val) | ((x == t_val) & (jc > t_idx))
                        better = elig & ((x > m_val)
                                         | ((x == m_val) & (jc < m_idx)))
                        m_val = jnp.where(better, x, m_val)
                        m_idx = jnp.where(better, jc, m_idx)
                    return m_val, m_idx

                t0 = (jnp.full((L,), jnp.inf, jnp.float32),
                      jnp.full((L,), -1, jnp.int32))
                t_val, t_idx = lax.fori_loop(0, NSEL, pass_body, t0)
                for j in range(W):
                    x = impv[j, sl]
                    jc = jnp.full((L,), j, jnp.int32)
                    sel = (x > t_val) | ((x == t_val) & (jc <= t_idx))
                    maskv[j, sl] = jnp.where(sel, jnp.float32(1.0),
                                             jnp.float32(0.0))
            pltpu.sync_copy(maskv, mask3_hbm.at[wid])

    return sc_mask, NWORK, QW


def _qkv_kernel(x_ref, gamma_ref, wqkv_ref, wcomb_ref, bcomb_ref,
                qkv_ref, gate_ref):
    x = x_ref[...]
    eps = jnp.finfo(jnp.float32).eps
    xn = x * jax.lax.rsqrt(jnp.mean(x * x, axis=-1, keepdims=True) + eps)
    xn = xn * gamma_ref[...]
    qkv_ref[...] = _dot(xn, wqkv_ref[...])
    gate_ref[...] = jax.nn.sigmoid(_dot(xn, wcomb_ref[...]) + bcomb_ref[...])


def _bc1_kernel(qw, kw_ref, vw_ref, kW1_ref, kb1_ref, kW2_ref, kb2_ref,
                vW1_ref, vb1_ref, vW2_ref, vb2_ref, memk_ref, memv_ref,
                q_ref, g_ref, ck_ref, cvb_ref, co_ref, impT_ref):
    i = pl.program_id(0)

    @pl.when(i == 0)
    def _b():
        hk = jnp.maximum(_dot(kw_ref[...], kW1_ref[...]) + kb1_ref[...], 0.0)
        ck = _dot(hk, kW2_ref[...]) + kb2_ref[...]
        hv = jnp.maximum(_dot(vw_ref[...], vW1_ref[...]) + vb1_ref[...], 0.0)
        cv = _dot(hv, vW2_ref[...]) + vb2_ref[...]
        ck_ref[...] = jnp.zeros((2 * W, DH), jnp.float32)
        cvb_ref[...] = jnp.zeros((2 * W, DH), jnp.bfloat16)
        ck_ref[0:W, :] = ck
        cvb_ref[0:W, :] = cv.astype(jnp.bfloat16)
        ck_ref[W:W + 1, :] = memk_ref[...]
        cvb_ref[W:W + 1, :] = memv_ref[...].astype(jnp.bfloat16)

    @pl.when(i > 0)
    def _c1():
        q = q_ref[...].reshape(H * TQ, DH)
        s = _dotT(q, ck_ref[...]) * SCALE
        col = jax.lax.broadcasted_iota(jnp.int32, s.shape, 1)
        sm = jnp.where(col <= W, s, NEG)
        e, rdenom = _softmax_unnorm(sm)
        co = (_dot(e, cvb_ref[...]) * rdenom * g_ref[...].reshape(H * TQ, 1)
              ).reshape(H, TQ, DH)
        co_ref[...] = co
        imp = jnp.mean(s.reshape(H, TQ, 2 * W)[:, :, 0:W], axis=0)
        for c in range(TQ // qw):
            impT_ref[c] = jnp.transpose(imp[c * qw:(c + 1) * qw, :])


def _mega2_kernel(qw, q_ref, maskT_ref, co_ref, kx_ref, vb_ref, kloc_ref,
                  vloc_ref, pos_ref, sigma_ref, g1_ref, g2_ref, wout_ref,
                  out_ref):
    q_all = q_ref[...]
    mtile = jnp.concatenate(
        [jnp.transpose(maskT_ref[c]) for c in range(TQ // qw)],
        axis=0) * MBIG

    pos = pos_ref[...]
    g2 = g2_ref[...]
    ball_outs = []
    for bi in range(NBT):
        pb = pos[bi * BALL:(bi + 1) * BALL]
        pt = jnp.transpose(pb)
        d2 = jnp.zeros((BALL, BALL), jnp.float32)
        for ci in range(3):
            diff = pb[:, ci:ci + 1] - pt[ci:ci + 1, :]
            d2 = d2 + diff * diff
        dist = jnp.sqrt(jnp.maximum(d2, 0.0))
        sig = jnp.broadcast_to(sigma_ref[...], (H, BALL, 1)
                               ).reshape(H * BALL, 1)
        bias = sig * jnp.broadcast_to(dist[None], (H, BALL, BALL)
                                      ).reshape(H * BALL, BALL)
        qb = q_all[:, bi * BALL:(bi + 1) * BALL, :].reshape(H * BALL, DH)
        sb = _dotT(qb * SCALE, kloc_ref[bi * BALL:(bi + 1) * BALL, :]) + bias
        eb, rdb = _softmax_unnorm(sb)
        ob = _dot(eb, vloc_ref[bi * BALL:(bi + 1) * BALL, :]) * rdb
        ob = ob * g2[:, bi * BALL:(bi + 1) * BALL, :].reshape(H * BALL, 1)
        ball_outs.append(ob.reshape(H, BALL, DH))
    so = jnp.concatenate(ball_outs, axis=1)

    g1 = g1_ref[...]
    co = co_ref[...]
    acc = jnp.zeros((TQ, D), jnp.float32)
    for hg in range(H // HB):
        qh = q_all[hg * HB:(hg + 1) * HB].reshape(HB * TQ, DH) * SCALE
        qa = jnp.concatenate(
            [qh, jnp.broadcast_to(mtile[None], (HB, TQ, W)
                                  ).reshape(HB * TQ, W)], axis=1)
        sf = _dotT(qa, kx_ref[...])
        ef, rdf = _softmax_unnorm(sf.reshape(HB, TQ, T))
        fine = _dot(ef.reshape(HB * TQ, T), vb_ref[...]).reshape(HB, TQ, DH)
        mixed = (fine * rdf * g1[hg * HB:(hg + 1) * HB]
                 + co[hg * HB:(hg + 1) * HB] + so[hg * HB:(hg + 1) * HB])
        for j in range(HB):
            hh = hg * HB + j
            acc = acc + _dot(mixed[j], wout_ref[hh * DH:(hh + 1) * DH, :])
    out_ref[...] = acc


def kernel(inp, pos, gamma, Wqkv, mem_kv, kW1, kb1, kW2, kb2,
           vW1, vb1, vW2, vb2, sigma_att, Wcomb, bcomb, Wout):
    x = inp.reshape(T, D)
    NQKV = H * DH + 2 * KVH * DH

    qkv, gate = pl.pallas_call(
        _qkv_kernel,
        grid=(NQT,),
        in_specs=[
            pl.BlockSpec((TQ, D), lambda i: (i, 0)),
            pl.BlockSpec((1, D), lambda i: (0, 0)),
            pl.BlockSpec((D, NQKV), lambda i: (0, 0)),
            pl.BlockSpec((D, 3 * H), lambda i: (0, 0)),
            pl.BlockSpec((1, 3 * H), lambda i: (0, 0)),
        ],
        out_specs=[
            pl.BlockSpec((TQ, NQKV), lambda i: (i, 0)),
            pl.BlockSpec((TQ, 3 * H), lambda i: (i, 0)),
        ],
        out_shape=[
            jax.ShapeDtypeStruct((T, NQKV), jnp.float32),
            jax.ShapeDtypeStruct((T, 3 * H), jnp.float32),
        ],
        interpret=_INTERPRET,
    )(x, gamma.reshape(1, D), Wqkv, Wcomb, bcomb.reshape(1, 3 * H))

    k = jax.lax.slice(qkv, (0, H * DH), (T, H * DH + DH))
    v = jax.lax.slice(qkv, (0, H * DH + DH), (T, H * DH + 2 * DH))
    kw = k.reshape(W, BLK * DH)
    vw = v.reshape(W, BLK * DH)
    vb = v.astype(jnp.bfloat16)
    expandT = (jnp.arange(T)[:, None] // BLK
               == jnp.arange(W)[None, :]).astype(jnp.float32)
    kx = jnp.concatenate([k, expandT], axis=1)
    q4 = qkv[:, :H * DH].reshape(T, H, DH).transpose(1, 0, 2)
    g0, g1, g2 = (gate[:, j::3].T.reshape(H, T, 1) for j in (0, 1, 2))

    sc_fn, NWORK, QW = _make_sc_mask()
    CPW = TQ // QW
    full = lambda shape: pl.BlockSpec(shape, lambda i: tuple(0 for _ in shape))
    tile = lambda i: (0, jnp.maximum(i - 1, 0), 0)
    _, _, co4, impT = pl.pallas_call(
        functools.partial(_bc1_kernel, QW),
        grid=(1 + NQT,),
        in_specs=[
            full((W, BLK * DH)), full((W, BLK * DH)),
            full((BLK * DH, BLK * DH)), full((1, BLK * DH)),
            full((BLK * DH, DH)), full((1, DH)),
            full((BLK * DH, BLK * DH)), full((1, BLK * DH)),
            full((BLK * DH, DH)), full((1, DH)),
            full((1, DH)), full((1, DH)),
            pl.BlockSpec((H, TQ, DH), tile),
            pl.BlockSpec((H, TQ, 1), tile),
        ],
        out_specs=[
            full((2 * W, DH)), full((2 * W, DH)),
            pl.BlockSpec((H, TQ, DH), tile),
            pl.BlockSpec((CPW, W, QW), lambda i: (jnp.maximum(i - 1, 0), 0, 0)),
        ],
        out_shape=[
            jax.ShapeDtypeStruct((2 * W, DH), jnp.float32),
            jax.ShapeDtypeStruct((2 * W, DH), jnp.bfloat16),
            jax.ShapeDtypeStruct((H, T, DH), jnp.float32),
            jax.ShapeDtypeStruct((NWORK, W, QW), jnp.float32),
        ],
        interpret=_INTERPRET,
    )(kw, vw, kW1, kb1.reshape(1, -1), kW2, kb2.reshape(1, -1),
      vW1, vb1.reshape(1, -1), vW2, vb2.reshape(1, -1),
      mem_kv[0, 0], mem_kv[1, 0], q4, g0)

    maskT = sc_fn(impT)

    posp = jnp.pad(pos, ((0, 0), (0, 8 - pos.shape[1])))
    sigma3 = sigma_att.reshape(H, 1, 1)
    out = pl.pallas_call(
        functools.partial(_mega2_kernel, QW),
        grid=(NQT,),
        in_specs=[
            pl.BlockSpec((H, TQ, DH), lambda i: (0, i, 0)),
            pl.BlockSpec((CPW, W, QW), lambda i: (i, 0, 0)),
            pl.BlockSpec((H, TQ, DH), lambda i: (0, i, 0)),
            pl.BlockSpec((T, DH + W), lambda i: (0, 0)),
            pl.BlockSpec((T, DH), lambda i: (0, 0)),
            pl.BlockSpec((TQ, DH), lambda i: (i, 0)),
            pl.BlockSpec((TQ, DH), lambda i: (i, 0)),
            pl.BlockSpec((TQ, 8), lambda i: (i, 0)),
            pl.BlockSpec((H, 1, 1), lambda i: (0, 0, 0)),
            pl.BlockSpec((H, TQ, 1), lambda i: (0, i, 0)),
            pl.BlockSpec((H, TQ, 1), lambda i: (0, i, 0)),
            pl.BlockSpec((H * DH, D), lambda i: (0, 0)),
        ],
        out_specs=pl.BlockSpec((TQ, D), lambda i: (i, 0)),
        out_shape=jax.ShapeDtypeStruct((T, D), jnp.float32),
        interpret=_INTERPRET,
    )(q4, maskT, co4, kx, vb, k, vb, posp, sigma3, g1, g2, Wout)

    return out.reshape(B, T, D)
